# baseline (speedup 1.0000x reference)
import functools

import jax
import jax.numpy as jnp
from jax import lax
from jax.experimental import pallas as pl
from jax.experimental.pallas import tpu as pltpu
from jax.experimental.pallas import tpu_sc as plsc

N = 16000
C = 81
C4 = 4 * C
EPS = 1e-7

NC, NS, L = 2, 16, 16
NW = NC * NS
NPAD = 16384
RPW = NPAD // NW
NBLK = 25
BN = N // NBLK


def _sigmoid(x):
    return 1.0 / (1.0 + jnp.exp(-x))


def _stats_body(ts_ref, os_ref, nl_ref, lab_ref, w_ref, w2_ref, aux_ref, wflat_ref, acc_ref):
    i = pl.program_id(0)
    ts = ts_ref[...]
    osv = os_ref[...]
    r = jnp.sum(osv, axis=1)
    p = jnp.sum(ts * osv, axis=1)
    cidx = lax.broadcasted_iota(jnp.int32, (BN, C), 1).astype(jnp.float32)
    labf = jnp.sum(ts * cidx, axis=1)
    lab_ref[pl.ds(i * BN, BN)] = labf.astype(jnp.int32)
    q = jnp.clip(p / r, EPS, 1.0 - EPS)
    nl_ref[pl.ds(i * BN, BN)] = -jnp.log(q)

    @pl.when(i == 0)
    def _():
        acc_ref[...] = jnp.zeros_like(acc_ref)

    acc_ref[0:1, 0:C] += jnp.sum(ts, axis=0, keepdims=True)

    @pl.when(i == NBLK - 1)
    def _():
        lab_ref[pl.ds(N, NPAD - N)] = jnp.zeros((NPAD - N,), jnp.int32)
        nl_ref[pl.ds(N, NPAD - N)] = jnp.zeros((NPAD - N,), jnp.float32)
        counts = acc_ref[...]
        ntot = jnp.sum(counts)
        npos = ntot - acc_ref[0, 0]
        w = _sigmoid(ntot / jnp.maximum(counts, EPS))
        w2 = _sigmoid(npos / jnp.maximum(counts, EPS))
        lane = lax.broadcasted_iota(jnp.int32, (1, 128), 1)
        w2 = jnp.where(lane == 0, 0.0, w2)
        w_ref[...] = w
        w2_ref[...] = w2
        wflat_ref[...] = w.reshape(128)
        inv_pos = 1.0 / jnp.maximum(EPS, npos)
        aux_ref[...] = jnp.full((1, 16), inv_pos, jnp.float32)


_stats_call = pl.pallas_call(
    _stats_body,
    grid=(NBLK,),
    in_specs=[
        pl.BlockSpec((BN, C), lambda i: (i, 0)),
        pl.BlockSpec((BN, C), lambda i: (i, 0)),
    ],
    out_specs=[
        pl.BlockSpec((NPAD,), lambda i: (0,)),
        pl.BlockSpec((NPAD,), lambda i: (0,)),
        pl.BlockSpec((1, 128), lambda i: (0, 0)),
        pl.BlockSpec((1, 128), lambda i: (0, 0)),
        pl.BlockSpec((1, 16), lambda i: (0, 0)),
        pl.BlockSpec((128,), lambda i: (0,)),
    ],
    out_shape=[
        jax.ShapeDtypeStruct((NPAD,), jnp.float32),
        jax.ShapeDtypeStruct((NPAD,), jnp.int32),
        jax.ShapeDtypeStruct((1, 128), jnp.float32),
        jax.ShapeDtypeStruct((1, 128), jnp.float32),
        jax.ShapeDtypeStruct((1, 16), jnp.float32),
        jax.ShapeDtypeStruct((128,), jnp.float32),
    ],
    scratch_shapes=[pltpu.VMEM((1, 128), jnp.float32)],
)


def _reg_body(od_ref, td_ref, ts_ref, w2_ref, aux_ref, out_ref, acc_ref):
    i = pl.program_id(0)
    od = od_ref[...]
    td = td_ref[...]
    ts = ts_ref[...]
    cc4 = lax.broadcasted_iota(jnp.int32, (C, C4), 1) >> 2
    cr = lax.broadcasted_iota(jnp.int32, (C, C4), 0)
    rmat = (cc4 == cr).astype(jnp.float32)
    w2v = w2_ref[...]
    w2rep = jnp.dot(w2v[:, :C], rmat)
    wfull = jnp.dot(ts, rmat) * w2rep
    d = jnp.abs(od - td) * wfull
    sl = jnp.where(d < 1.0, 0.5 * d * d, d - 0.5)

    @pl.when(i == 0)
    def _():
        acc_ref[0, 0] = 0.0

    acc_ref[0, 0] += jnp.sum(sl)

    @pl.when(i == NBLK - 1)
    def _():
        out_ref[0, 0] = acc_ref[0, 0] * aux_ref[0, 0]


_reg_call = pl.pallas_call(
    _reg_body,
    grid=(NBLK,),
    in_specs=[
        pl.BlockSpec((BN, C4), lambda i: (i, 0)),
        pl.BlockSpec((BN, C4), lambda i: (i, 0)),
        pl.BlockSpec((BN, C), lambda i: (i, 0)),
        pl.BlockSpec((1, 128), lambda i: (0, 0)),
        pl.BlockSpec((1, 16), lambda i: (0, 0)),
    ],
    out_specs=pl.BlockSpec(memory_space=pltpu.SMEM),
    out_shape=jax.ShapeDtypeStruct((1, 1), jnp.float32),
    scratch_shapes=[pltpu.SMEM((1, 1), jnp.float32)],
)


_sc_mesh = plsc.VectorSubcoreMesh(core_axis_name="c", subcore_axis_name="s")


@functools.partial(
    pl.kernel,
    out_type=jax.ShapeDtypeStruct((NW, 16), jnp.float32),
    mesh=_sc_mesh,
    scratch_types=[
        pltpu.VMEM((RPW,), jnp.int32),
        pltpu.VMEM((RPW,), jnp.float32),
        pltpu.VMEM((128,), jnp.float32),
        pltpu.VMEM((16,), jnp.float32),
    ],
    compiler_params=pltpu.CompilerParams(
        needs_layout_passes=False, use_tc_tiling_on_sc=False
    ),
)
def _cls_call(lab_hbm, nl_hbm, w_hbm, out_hbm, lab_v, nl_v, w_v, out_v):
    wid = lax.axis_index("s") * NC + lax.axis_index("c")
    base = wid * RPW
    pltpu.sync_copy(lab_hbm.at[pl.ds(base, RPW)], lab_v)
    pltpu.sync_copy(nl_hbm.at[pl.ds(base, RPW)], nl_v)
    pltpu.sync_copy(w_hbm, w_v)

    cacc = jnp.zeros((L,), jnp.float32)
    for m in range(RPW // L):
        lab16 = lab_v[pl.ds(m * L, L)]
        wv = plsc.load_gather(w_v, [lab16])
        cacc = cacc + nl_v[pl.ds(m * L, L)] * wv

    out_v[...] = cacc * (1.0 / N)
    pltpu.sync_copy(out_v, out_hbm.at[wid])


def kernel(target_deltas, target_scores, output_deltas, output_scores):
    ts2 = target_scores.reshape(N, C)
    os2 = output_scores.reshape(N, C)
    nl, lab, w, w2, aux, wflat = _stats_call(ts2, os2)
    od2 = output_deltas.reshape(N, C4)
    td2 = target_deltas.reshape(N, C4)
    reg = _reg_call(od2, td2, ts2, w2, aux)
    cls_parts = _cls_call(lab, nl, wflat)
    return jnp.sum(cls_parts) + reg[0, 0]


# speedup vs baseline: 1.1427x; 1.1427x over previous
import functools

import jax
import jax.numpy as jnp
from jax import lax
from jax.experimental import pallas as pl
from jax.experimental.pallas import tpu as pltpu
from jax.experimental.pallas import tpu_sc as plsc

N = 16000
C = 81
C4 = 4 * C
EPS = 1e-7

NC, NS, L = 2, 16, 16
NW = NC * NS
NPAD = 16384
RPW = NPAD // NW
NBLK = 5
BN = N // NBLK


def _sigmoid(x):
    return 1.0 / (1.0 + jnp.exp(-x))


def _stats_body(ts_ref, os_ref, nl_ref, lab_ref, w_ref, w2_ref, aux_ref, wflat_ref, acc_ref):
    i = pl.program_id(0)
    ts = ts_ref[...]
    osv = os_ref[...]
    r = jnp.sum(osv, axis=1)
    p = jnp.sum(ts * osv, axis=1)
    cidx = lax.broadcasted_iota(jnp.int32, (BN, C), 1).astype(jnp.float32)
    labf = jnp.sum(ts * cidx, axis=1)
    lab_ref[pl.ds(i * BN, BN)] = labf.astype(jnp.int32)
    q = jnp.clip(p / r, EPS, 1.0 - EPS)
    nl_ref[pl.ds(i * BN, BN)] = -jnp.log(q)

    @pl.when(i == 0)
    def _():
        acc_ref[...] = jnp.zeros_like(acc_ref)

    acc_ref[0:1, 0:C] += jnp.sum(ts, axis=0, keepdims=True)

    @pl.when(i == NBLK - 1)
    def _():
        lab_ref[pl.ds(N, NPAD - N)] = jnp.zeros((NPAD - N,), jnp.int32)
        nl_ref[pl.ds(N, NPAD - N)] = jnp.zeros((NPAD - N,), jnp.float32)
        counts = acc_ref[...]
        ntot = jnp.sum(counts)
        npos = ntot - acc_ref[0, 0]
        w = _sigmoid(ntot / jnp.maximum(counts, EPS))
        w2 = _sigmoid(npos / jnp.maximum(counts, EPS))
        lane = lax.broadcasted_iota(jnp.int32, (1, 128), 1)
        w2 = jnp.where(lane == 0, 0.0, w2)
        w_ref[...] = w
        w2_ref[...] = w2
        wflat_ref[...] = w.reshape(128)
        inv_pos = 1.0 / jnp.maximum(EPS, npos)
        aux_ref[...] = jnp.full((1, 16), inv_pos, jnp.float32)


_stats_call = pl.pallas_call(
    _stats_body,
    grid=(NBLK,),
    in_specs=[
        pl.BlockSpec((BN, C), lambda i: (i, 0)),
        pl.BlockSpec((BN, C), lambda i: (i, 0)),
    ],
    out_specs=[
        pl.BlockSpec((NPAD,), lambda i: (0,)),
        pl.BlockSpec((NPAD,), lambda i: (0,)),
        pl.BlockSpec((1, 128), lambda i: (0, 0)),
        pl.BlockSpec((1, 128), lambda i: (0, 0)),
        pl.BlockSpec((1, 16), lambda i: (0, 0)),
        pl.BlockSpec((128,), lambda i: (0,)),
    ],
    out_shape=[
        jax.ShapeDtypeStruct((NPAD,), jnp.float32),
        jax.ShapeDtypeStruct((NPAD,), jnp.int32),
        jax.ShapeDtypeStruct((1, 128), jnp.float32),
        jax.ShapeDtypeStruct((1, 128), jnp.float32),
        jax.ShapeDtypeStruct((1, 16), jnp.float32),
        jax.ShapeDtypeStruct((128,), jnp.float32),
    ],
    scratch_shapes=[pltpu.VMEM((1, 128), jnp.float32)],
)


def _reg_body(od_ref, td_ref, ts_ref, w2_ref, aux_ref, out_ref, acc_ref):
    i = pl.program_id(0)
    od = od_ref[...]
    td = td_ref[...]
    ts = ts_ref[...]
    cc4 = lax.broadcasted_iota(jnp.int32, (C, C4), 1) >> 2
    cr = lax.broadcasted_iota(jnp.int32, (C, C4), 0)
    rmat = (cc4 == cr).astype(jnp.float32)
    w2v = w2_ref[...]
    w2rep = jnp.dot(w2v[:, :C], rmat)
    wfull = jnp.dot(ts, rmat) * w2rep
    d = jnp.abs(od - td) * wfull
    sl = jnp.where(d < 1.0, 0.5 * d * d, d - 0.5)

    @pl.when(i == 0)
    def _():
        acc_ref[0, 0] = 0.0

    acc_ref[0, 0] += jnp.sum(sl)

    @pl.when(i == NBLK - 1)
    def _():
        out_ref[0, 0] = acc_ref[0, 0] * aux_ref[0, 0]


_reg_call = pl.pallas_call(
    _reg_body,
    grid=(NBLK,),
    in_specs=[
        pl.BlockSpec((BN, C4), lambda i: (i, 0)),
        pl.BlockSpec((BN, C4), lambda i: (i, 0)),
        pl.BlockSpec((BN, C), lambda i: (i, 0)),
        pl.BlockSpec((1, 128), lambda i: (0, 0)),
        pl.BlockSpec((1, 16), lambda i: (0, 0)),
    ],
    out_specs=pl.BlockSpec(memory_space=pltpu.SMEM),
    out_shape=jax.ShapeDtypeStruct((1, 1), jnp.float32),
    scratch_shapes=[pltpu.SMEM((1, 1), jnp.float32)],
)


_sc_mesh = plsc.VectorSubcoreMesh(core_axis_name="c", subcore_axis_name="s")


@functools.partial(
    pl.kernel,
    out_type=jax.ShapeDtypeStruct((NW, 16), jnp.float32),
    mesh=_sc_mesh,
    scratch_types=[
        pltpu.VMEM((RPW,), jnp.int32),
        pltpu.VMEM((RPW,), jnp.float32),
        pltpu.VMEM((128,), jnp.float32),
        pltpu.VMEM((16,), jnp.float32),
    ],
    compiler_params=pltpu.CompilerParams(
        needs_layout_passes=False, use_tc_tiling_on_sc=False
    ),
)
def _cls_call(lab_hbm, nl_hbm, w_hbm, out_hbm, lab_v, nl_v, w_v, out_v):
    wid = lax.axis_index("s") * NC + lax.axis_index("c")
    base = wid * RPW
    pltpu.sync_copy(lab_hbm.at[pl.ds(base, RPW)], lab_v)
    pltpu.sync_copy(nl_hbm.at[pl.ds(base, RPW)], nl_v)
    pltpu.sync_copy(w_hbm, w_v)

    cacc = jnp.zeros((L,), jnp.float32)
    for m in range(RPW // L):
        lab16 = lab_v[pl.ds(m * L, L)]
        wv = plsc.load_gather(w_v, [lab16])
        cacc = cacc + nl_v[pl.ds(m * L, L)] * wv

    out_v[...] = cacc * (1.0 / N)
    pltpu.sync_copy(out_v, out_hbm.at[wid])


def kernel(target_deltas, target_scores, output_deltas, output_scores):
    ts2 = target_scores.reshape(N, C)
    os2 = output_scores.reshape(N, C)
    nl, lab, w, w2, aux, wflat = _stats_call(ts2, os2)
    od2 = output_deltas.reshape(N, C4)
    td2 = target_deltas.reshape(N, C4)
    reg = _reg_call(od2, td2, ts2, w2, aux)
    cls_parts = _cls_call(lab, nl, wflat)
    return jnp.sum(cls_parts) + reg[0, 0]
